# Initial kernel scaffold; baseline (speedup 1.0000x reference)
import jax
import jax.numpy as jnp
from jax.experimental import pallas as pl


def kernel(pred_batch, doa_gt, vad_gt, tmpl_re, tmpl_im, azi_grid):
    raise NotImplementedError("write your pallas kernel here")


# speedup vs baseline: 1.8174x; 1.8174x over previous
import functools

import jax
import jax.numpy as jnp
from jax.experimental import pallas as pl

NB, NT, NF, NAZI = 32, 256, 256, 180
ROWS = NB * NT
TILE = 512
RAD2DEG = 180.0 / 3.141592653589793


def _fused_kernel(x_ref, w_ref, azi_ref, azigt_ref, vad_ref,
                  ss_ref, doa_ref, acc_ref):
    i = pl.program_id(0)
    x = x_ref[...]
    w = w_ref[...]
    ss = jnp.dot(x, w, preferred_element_type=jnp.float32)
    ss_ref[...] = ss

    azi = azi_ref[...]
    mx = jnp.max(ss, axis=1, keepdims=True)
    hit = ss >= mx
    doa = jnp.min(jnp.where(hit, azi, jnp.inf), axis=1, keepdims=True)
    doa_ref[...] = doa

    azi_gt = azigt_ref[...]
    vad = (vad_ref[...] > 0.5).astype(jnp.float32)
    err = jnp.abs(doa - azi_gt) * RAD2DEG
    err = jnp.minimum(err, 360.0 - err)
    corr = (err < 30.0).astype(jnp.float32) * vad
    lane = jax.lax.broadcasted_iota(jnp.int32, (1, 128), 1)
    part = (jnp.where(lane == 0, jnp.sum(corr), 0.0)
            + jnp.where(lane == 1, jnp.sum(vad * err), 0.0)
            + jnp.where(lane == 2, jnp.sum(vad), 0.0))

    @pl.when(i == 0)
    def _init():
        acc_ref[...] = part

    @pl.when(i > 0)
    def _accum():
        acc_ref[...] += part


@functools.partial(jax.jit, static_argnames=())
def kernel(pred_batch, doa_gt, vad_gt, tmpl_re, tmpl_im, azi_grid):
    x = pred_batch.reshape(ROWS, 2 * NF)
    w = jnp.concatenate([tmpl_re.T, tmpl_im.T], axis=0)
    azi2 = azi_grid.reshape(1, NAZI)
    azi_gt = doa_gt[:, :, 1, 0].reshape(ROWS, 1)
    vad2 = vad_gt.reshape(ROWS, 1)

    grid = (ROWS // TILE,)
    ss, doa, acc = pl.pallas_call(
        _fused_kernel,
        grid=grid,
        in_specs=[
            pl.BlockSpec((TILE, 2 * NF), lambda i: (i, 0)),
            pl.BlockSpec((2 * NF, NAZI), lambda i: (0, 0)),
            pl.BlockSpec((1, NAZI), lambda i: (0, 0)),
            pl.BlockSpec((TILE, 1), lambda i: (i, 0)),
            pl.BlockSpec((TILE, 1), lambda i: (i, 0)),
        ],
        out_specs=[
            pl.BlockSpec((TILE, NAZI), lambda i: (i, 0)),
            pl.BlockSpec((TILE, 1), lambda i: (i, 0)),
            pl.BlockSpec((1, 128), lambda i: (0, 0)),
        ],
        out_shape=[
            jax.ShapeDtypeStruct((ROWS, NAZI), jnp.float32),
            jax.ShapeDtypeStruct((ROWS, 1), jnp.float32),
            jax.ShapeDtypeStruct((1, 128), jnp.float32),
        ],
    )(x, w, azi2, azi_gt, vad2)

    ss = ss.reshape(NB, NT, NAZI)
    doa_est_azi = doa.reshape(NB, NT)
    denom = jnp.maximum(acc[0, 2], 1.0)
    ACC = acc[0, 0] / denom
    MAE = acc[0, 1] / denom
    return ss, doa_est_azi, ACC, MAE
